# baseline (speedup 1.0000x reference)
import functools

import jax
import jax.numpy as jnp
from jax import lax
from jax.experimental import pallas as pl
from jax.experimental.pallas import tpu as pltpu
from jax.experimental.pallas import tpu_sc as plsc

SRC = 2048
BATCH = 4
HIDDEN = 768
ROWS = SRC * BATCH
NC, NS = 2, 16
NW = NC * NS
NHALF = 2
ROWS_H = ROWS // NHALF
SRC_H = SRC // NHALF
R_PER_W = ROWS_H // NW
CHUNK = 64
NCHUNK = R_PER_W // CHUNK


def _gather_sc(table, ids_half):
    mesh = plsc.VectorSubcoreMesh(core_axis_name="c", subcore_axis_name="s")

    @functools.partial(
        pl.kernel,
        mesh=mesh,
        out_type=jax.ShapeDtypeStruct((ROWS_H, HIDDEN), jnp.float32),
        scratch_types=[
            pltpu.VMEM((R_PER_W,), jnp.int32),
            pltpu.VMEM((CHUNK, HIDDEN), jnp.float32),
            pltpu.VMEM((CHUNK, HIDDEN), jnp.float32),
            pltpu.SemaphoreType.DMA,
            pltpu.SemaphoreType.DMA,
        ],
    )
    def gather_kernel(table_hbm, idx_hbm, out_hbm, idx_v, buf0, buf1,
                      sem0, sem1):
        wid = lax.axis_index("s") * NC + lax.axis_index("c")
        base = wid * R_PER_W
        bufs, sems = (buf0, buf1), (sem0, sem1)
        pltpu.sync_copy(idx_hbm.at[pl.ds(base, R_PER_W)], idx_v)
        cps = [None] * NCHUNK
        cps[0] = pltpu.async_copy(
            table_hbm.at[idx_v.at[pl.ds(0, CHUNK)]], buf0, sem0)
        for c in range(NCHUNK):
            cps[c].wait()
            if c + 1 < NCHUNK:
                cps[c + 1] = pltpu.async_copy(
                    table_hbm.at[idx_v.at[pl.ds((c + 1) * CHUNK, CHUNK)]],
                    bufs[(c + 1) % 2], sems[(c + 1) % 2])
            pltpu.sync_copy(bufs[c % 2],
                            out_hbm.at[pl.ds(base + c * CHUNK, CHUNK)])

    return gather_kernel(table, ids_half)


BS_S = 128
BLK = BS_S * BATCH


def _tc_fused_half(tok_half, pos3, w_t, b2, g2, bt2, half, prev_out):
    nblk = SRC_H // BS_S
    off = half * nblk

    def body(*refs):
        if half > 0:
            refs = refs[1:]
        tok_ref, pos_ref, w_ref, b_ref, g_ref, bt_ref, out_ref = refs
        pos = pos_ref[...].reshape(BLK, HIDDEN)
        acc = jnp.dot(pos, w_ref[...], preferred_element_type=jnp.float32)
        p = 1.0 / (1.0 + jnp.exp(-(acc + b_ref[...])))
        e = tok_ref[...] + p
        mean = jnp.mean(e, axis=1, keepdims=True)
        cen = e - mean
        var = jnp.mean(cen * cen, axis=1, keepdims=True)
        res = cen * lax.rsqrt(var + 1e-5) * g_ref[...] + bt_ref[...]
        out_ref[...] = res.reshape(BS_S, BATCH, HIDDEN)

    in_specs = [
        pl.BlockSpec((BLK, HIDDEN), lambda i: (i, 0)),
        pl.BlockSpec((BS_S, BATCH, HIDDEN), lambda i: (i + off, 0, 0)),
        pl.BlockSpec((HIDDEN, HIDDEN), lambda i: (0, 0)),
        pl.BlockSpec((1, HIDDEN), lambda i: (0, 0)),
        pl.BlockSpec((1, HIDDEN), lambda i: (0, 0)),
        pl.BlockSpec((1, HIDDEN), lambda i: (0, 0)),
    ]
    args = [tok_half, pos3, w_t, b2, g2, bt2]
    aliases = {}
    if half > 0:
        in_specs = [pl.BlockSpec(memory_space=pl.ANY)] + in_specs
        args = [prev_out] + args
        aliases = {0: 0}

    return pl.pallas_call(
        body,
        grid=(nblk,),
        in_specs=in_specs,
        out_specs=pl.BlockSpec((BS_S, BATCH, HIDDEN),
                               lambda i: (i + off, 0, 0)),
        out_shape=jax.ShapeDtypeStruct((SRC, BATCH, HIDDEN), jnp.float32),
        input_output_aliases=aliases,
    )(*args)


def kernel(input_ids, position_ids, W_tok, W_pd, b_pd, gamma, beta):
    ids_flat = input_ids.reshape(ROWS).astype(jnp.int32)
    w_t = W_pd.T
    b2 = b_pd.reshape(1, HIDDEN)
    g2 = gamma.reshape(1, HIDDEN)
    bt2 = beta.reshape(1, HIDDEN)

    toks = [
        _gather_sc(W_tok, lax.slice(ids_flat, (h * ROWS_H,),
                                    ((h + 1) * ROWS_H,)))
        for h in range(NHALF)
    ]
    out = None
    for h in range(NHALF):
        out = _tc_fused_half(toks[h], position_ids, w_t, b2, g2, bt2,
                             h, out)
    return out
